# baseline (speedup 1.0000x reference)
import jax
import jax.numpy as jnp
from jax.experimental import pallas as pl

_NUM_EMB = 1024
_DIM = 64
_BLK = 256


def _vq_block(x_ref, cb_ref, enc_ref, q_ref):
    x = x_ref[...]
    cb = cb_ref[...]
    scores = jax.lax.dot_general(
        x, cb, (((1,), (1,)), ((), ())), preferred_element_type=jnp.float32
    )
    x2 = jnp.sum(x * x, axis=1, keepdims=True)
    e2_col = jnp.sum(cb * cb, axis=1, keepdims=True)
    e2_row = jax.lax.transpose(e2_col, (1, 0))
    dist = (x2 + e2_row) - 2.0 * scores
    min_d = jnp.min(dist, axis=1, keepdims=True)
    iota = jax.lax.broadcasted_iota(jnp.int32, dist.shape, 1)
    cand = jnp.where(dist <= min_d, iota, jnp.int32(_NUM_EMB))
    idx = jnp.min(cand, axis=1, keepdims=True)
    enc = (iota == idx).astype(jnp.float32)
    enc_ref[...] = enc
    q_ref[...] = jax.lax.dot_general(
        enc, cb, (((1,), (0,)), ((), ())),
        preferred_element_type=jnp.float32,
        precision=jax.lax.Precision.HIGHEST,
    )


def kernel(inputs, codebook):
    input_shape = inputs.shape
    flat = inputs.reshape(-1, _DIM)
    n = flat.shape[0]
    grid = n // _BLK

    enc, quant = pl.pallas_call(
        _vq_block,
        grid=(grid,),
        in_specs=[
            pl.BlockSpec((_BLK, _DIM), lambda i: (i, 0)),
            pl.BlockSpec((_NUM_EMB, _DIM), lambda i: (0, 0)),
        ],
        out_specs=[
            pl.BlockSpec((_BLK, _NUM_EMB), lambda i: (i, 0)),
            pl.BlockSpec((_BLK, _DIM), lambda i: (i, 0)),
        ],
        out_shape=[
            jax.ShapeDtypeStruct((n, _NUM_EMB), jnp.float32),
            jax.ShapeDtypeStruct((n, _DIM), jnp.float32),
        ],
    )(flat, codebook)

    return quant.reshape(input_shape), enc


# speedup vs baseline: 1.4865x; 1.4865x over previous
import jax
import jax.numpy as jnp
from jax.experimental import pallas as pl

_NUM_EMB = 1024
_DIM = 64
_BLK = 256


def _prep_block(cb_ref, e2_ref, m2cb_ref):
    cb = cb_ref[...]
    e2_col = jnp.sum(cb * cb, axis=1, keepdims=True)
    e2_ref[...] = jax.lax.transpose(e2_col, (1, 0))
    m2cb_ref[...] = cb * -2.0


def _vq_block(x_ref, cb_ref, e2_ref, m2cb_ref, enc_ref, q_ref):
    x = x_ref[...]
    scores2 = jax.lax.dot_general(
        x, m2cb_ref[...], (((1,), (1,)), ((), ())),
        preferred_element_type=jnp.float32,
    )
    x2 = jnp.sum(x * x, axis=1, keepdims=True)
    dist = (x2 + e2_ref[...]) + scores2
    min_d = jnp.min(dist, axis=1, keepdims=True)
    iota = jax.lax.broadcasted_iota(jnp.int32, dist.shape, 1)
    cand = jnp.where(dist <= min_d, iota, jnp.int32(_NUM_EMB))
    idx = jnp.min(cand, axis=1, keepdims=True)
    enc = (iota == idx).astype(jnp.float32)
    enc_ref[...] = enc
    q_ref[...] = jax.lax.dot_general(
        enc, cb_ref[...], (((1,), (0,)), ((), ())),
        preferred_element_type=jnp.float32,
    )


def kernel(inputs, codebook):
    input_shape = inputs.shape
    flat = inputs.reshape(-1, _DIM)
    n = flat.shape[0]
    grid = n // _BLK

    e2_row, m2cb = pl.pallas_call(
        _prep_block,
        out_shape=[
            jax.ShapeDtypeStruct((1, _NUM_EMB), jnp.float32),
            jax.ShapeDtypeStruct((_NUM_EMB, _DIM), jnp.float32),
        ],
    )(codebook)

    enc, quant = pl.pallas_call(
        _vq_block,
        grid=(grid,),
        in_specs=[
            pl.BlockSpec((_BLK, _DIM), lambda i: (i, 0)),
            pl.BlockSpec((_NUM_EMB, _DIM), lambda i: (0, 0)),
            pl.BlockSpec((1, _NUM_EMB), lambda i: (0, 0)),
            pl.BlockSpec((_NUM_EMB, _DIM), lambda i: (0, 0)),
        ],
        out_specs=[
            pl.BlockSpec((_BLK, _NUM_EMB), lambda i: (i, 0)),
            pl.BlockSpec((_BLK, _DIM), lambda i: (i, 0)),
        ],
        out_shape=[
            jax.ShapeDtypeStruct((n, _NUM_EMB), jnp.float32),
            jax.ShapeDtypeStruct((n, _DIM), jnp.float32),
        ],
    )(flat, codebook, e2_row, m2cb)

    return quant.reshape(input_shape), enc
